# baseline (speedup 1.0000x reference)
import functools

import jax
import jax.numpy as jnp
from jax import lax
from jax.experimental import pallas as pl
from jax.experimental.pallas import tpu as pltpu
from jax.experimental.pallas import tpu_sc as plsc

NUM_EXPERTS = 16
HIDDEN = 1024
MOE_FF = 512
TM = 128
NT = 48
S_PAD = NT * TM
NW = 32


def _router_dispatch(x_ref, gate_ref, d0_ref, d1_ref, w0_ref, w1_ref,
                     te_ref, xb_ref, ac_ref, mt_ref, r1_ref, r2_ref):
    T = x_ref.shape[0]
    E = NUM_EXPERTS
    x = x_ref[...]
    logits = lax.dot_general(x, gate_ref[...], (((1,), (1,)), ((), ())),
                             preferred_element_type=jnp.float32)
    m = jnp.max(logits, axis=1, keepdims=True)
    ex = jnp.exp(logits - m)
    sm = ex / jnp.sum(ex, axis=1, keepdims=True)
    lane = lax.broadcasted_iota(jnp.int32, sm.shape, 1)
    m1 = jnp.max(sm, axis=1, keepdims=True)
    i1 = jnp.min(jnp.where(sm == m1, lane, E), axis=1, keepdims=True)
    oh1 = (lane == i1).astype(jnp.float32)
    smm = jnp.where(lane == i1, -jnp.inf, sm)
    m2 = jnp.max(smm, axis=1, keepdims=True)
    i2 = jnp.min(jnp.where(smm == m2, lane, E), axis=1, keepdims=True)
    oh2 = (lane == i2).astype(jnp.float32)
    denom = m1 + m2
    w0_ref[...] = m1 / denom
    w1_ref[...] = m2 / denom

    CH = 128
    rr = lax.broadcasted_iota(jnp.int32, (CH, CH), 0)
    cc = lax.broadcasted_iota(jnp.int32, (CH, CH), 1)
    lex = (cc < rr).astype(jnp.float32)
    off1 = jnp.zeros((1, E), jnp.float32)
    off2 = jnp.zeros((1, E), jnp.float32)
    for c in range(T // CH):
        sl = slice(c * CH, (c + 1) * CH)
        o1c = oh1[sl, :]
        o2c = oh2[sl, :]
        r1_ref[sl, :] = off1 + lax.dot_general(
            lex, o1c, (((1,), (0,)), ((), ())),
            preferred_element_type=jnp.float32)
        r2_ref[sl, :] = off2 + lax.dot_general(
            lex, o2c, (((1,), (0,)), ((), ())),
            preferred_element_type=jnp.float32)
        off1 = off1 + jnp.sum(o1c, axis=0, keepdims=True)
        off2 = off2 + jnp.sum(o2c, axis=0, keepdims=True)
    counts1 = off1
    counts = off1 + off2
    tcnt = jnp.floor((counts + (TM - 1)) * (1.0 / TM))
    er = lax.broadcasted_iota(jnp.int32, (E, E), 0)
    ec = lax.broadcasted_iota(jnp.int32, (E, E), 1)
    strict_upper = (er < ec).astype(jnp.float32)
    ts = lax.dot_general(tcnt, strict_upper, (((1,), (0,)), ((), ())),
                         preferred_element_type=jnp.float32)
    pt_start = ts * TM
    total = jnp.sum(tcnt)

    d0 = jnp.sum(oh1 * (pt_start + r1_ref[...]), axis=1, keepdims=True)
    d1 = jnp.sum(oh2 * (pt_start + counts1 + r2_ref[...]), axis=1,
                 keepdims=True)
    d0_ref[...] = d0.astype(jnp.int32)
    d1_ref[...] = d1.astype(jnp.int32)

    jcol = lax.broadcasted_iota(jnp.int32, (NT, 1), 0).astype(jnp.float32)
    jj = jnp.minimum(jcol, total - 1.0)
    lane_e = lax.broadcasted_iota(jnp.int32, (NT, E), 1).astype(jnp.float32)
    in_e = ((jj >= ts) & (jj < ts + tcnt)).astype(jnp.float32)
    te_ref[...] = jnp.sum(lane_e * in_e, axis=1, keepdims=True).astype(jnp.int32)
    xb_ref[...] = jj.astype(jnp.int32)
    ac_ref[...] = (jcol < total).astype(jnp.int32)
    mt_ref[...] = jnp.full((16, 1), TM, jnp.float32).astype(jnp.int32) * (
        total.astype(jnp.int32))


def _sc_dispatch_gather(x_hbm, d0_hbm, d1_hbm, w0_hbm, w1_hbm, mt_hbm,
                        xs_hbm, ws_hbm,
                        dest_v, wval_v, tok_v, wsort_v, meta_v, buf0_v, buf1_v,
                        sem_i0, sem_i1, sem_o0, sem_o1):
    T = d0_hbm.shape[0]
    wid = lax.axis_index("s") * 2 + lax.axis_index("c")
    pltpu.sync_copy(mt_hbm, meta_v)

    zf = jnp.zeros((16,), jnp.float32)

    def zbody(i, _):
        tok_v[pl.ds(i * 16, 16)] = (i * 16 + lax.iota(jnp.int32, 16)) & (T - 1)
        wsort_v[pl.ds(i * 16, 16)] = zf
        return 0

    lax.fori_loop(0, S_PAD // 16, zbody, 0, unroll=4)

    for d_hbm, w_hbm in ((d0_hbm, w0_hbm), (d1_hbm, w1_hbm)):
        pltpu.sync_copy(d_hbm, dest_v)
        pltpu.sync_copy(w_hbm, wval_v)

        def sbody(j, _):
            idx = dest_v[pl.ds(j * 16, 16)]
            vals = j * 16 + lax.iota(jnp.int32, 16)
            plsc.store_scatter(tok_v, [idx], vals)
            wv = wval_v[pl.ds(j * 16, 16)]
            plsc.store_scatter(wsort_v, [idx], wv)
            return 0

        lax.fori_loop(0, T // 16, sbody, 0, unroll=4)

    rows = S_PAD // NW
    base = pl.multiple_of(wid * rows, 8)
    pltpu.sync_copy(wsort_v.at[pl.ds(base, rows)],
                    ws_hbm.at[pl.ds(base, rows)])
    total_rows = meta_v[pl.ds(0, 16)][0]

    @pl.when(base < total_rows)
    def _():
        CH = 48
        nchk = rows // CH
        bufs = (buf0_v, buf1_v)
        sin = (sem_i0, sem_i1)
        sout = (sem_o0, sem_o1)

        def sl(c):
            return pl.ds(base + c * CH, CH)

        incp = [None] * nchk
        outcp = [None] * nchk
        incp[0] = pltpu.async_copy(x_hbm.at[tok_v.at[sl(0)]], bufs[0], sin[0])
        for c in range(nchk):
            if c + 1 < nchk:
                if c >= 1:
                    outcp[c - 1].wait()
                incp[c + 1] = pltpu.async_copy(
                    x_hbm.at[tok_v.at[sl(c + 1)]], bufs[(c + 1) % 2],
                    sin[(c + 1) % 2])
            incp[c].wait()
            outcp[c] = pltpu.async_copy(bufs[c % 2], xs_hbm.at[sl(c)],
                                        sout[c % 2])
        outcp[nchk - 2].wait()
        outcp[nchk - 1].wait()


def _moe_mm(te_ref, xb_ref, ac_ref, xs_ref, w1_ref, w2_ref, w3_ref, ws_ref,
            y_ref):
    j = pl.program_id(0)

    @pl.when(ac_ref[j] == 1)
    def _():
        xb = xs_ref[...].astype(jnp.bfloat16)
        h = lax.dot_general(xb, w1_ref[0].astype(jnp.bfloat16),
                            (((1,), (1,)), ((), ())),
                            preferred_element_type=jnp.float32)
        u = lax.dot_general(xb, w3_ref[0].astype(jnp.bfloat16),
                            (((1,), (1,)), ((), ())),
                            preferred_element_type=jnp.float32)
        act = (h * jax.nn.sigmoid(h) * u).astype(jnp.bfloat16)
        y = lax.dot_general(act, w2_ref[0].astype(jnp.bfloat16),
                            (((1,), (1,)), ((), ())),
                            preferred_element_type=jnp.float32)
        y_ref[...] = y * ws_ref[...]


def _sc_combine_gather(y_hbm, d0_hbm, d1_hbm, g0_hbm, g1_hbm,
                       idx_v, buf0_v, buf1_v, sem_i0, sem_i1, sem_o0, sem_o1):
    T = d0_hbm.shape[0]
    wid = lax.axis_index("s") * 2 + lax.axis_index("c")
    tpw = T // NW
    base = pl.multiple_of(wid * tpw, 8)
    pltpu.sync_copy(d0_hbm.at[pl.ds(base, tpw)], idx_v.at[pl.ds(0, tpw)])
    pltpu.sync_copy(d1_hbm.at[pl.ds(base, tpw)], idx_v.at[pl.ds(tpw, tpw)])
    CH = 32
    nchk = 2 * tpw // CH
    bufs = (buf0_v, buf1_v)
    sin = (sem_i0, sem_i1)
    sout = (sem_o0, sem_o1)
    outs = []
    for k in range(nchk):
        g_hbm = g0_hbm if k < nchk // 2 else g1_hbm
        outs.append(g_hbm.at[pl.ds(base + (k % (nchk // 2)) * CH, CH)])

    incp = [None] * nchk
    outcp = [None] * nchk
    incp[0] = pltpu.async_copy(y_hbm.at[idx_v.at[pl.ds(0, CH)]],
                               bufs[0], sin[0])
    for c in range(nchk):
        if c + 1 < nchk:
            if c >= 1:
                outcp[c - 1].wait()
            incp[c + 1] = pltpu.async_copy(
                y_hbm.at[idx_v.at[pl.ds((c + 1) * CH, CH)]],
                bufs[(c + 1) % 2], sin[(c + 1) % 2])
        incp[c].wait()
        outcp[c] = pltpu.async_copy(bufs[c % 2], outs[c], sout[c % 2])
    outcp[nchk - 2].wait()
    outcp[nchk - 1].wait()


def _shared_final(x_ref, ws1_ref, ws2_ref, ws3_ref, sg_ref, g0_ref, g1_ref,
                  out_ref):
    x = x_ref[...]
    xb = x.astype(jnp.bfloat16)
    h = lax.dot_general(xb, ws1_ref[...].astype(jnp.bfloat16),
                        (((1,), (1,)), ((), ())),
                        preferred_element_type=jnp.float32)
    u = lax.dot_general(xb, ws3_ref[...].astype(jnp.bfloat16),
                        (((1,), (1,)), ((), ())),
                        preferred_element_type=jnp.float32)
    act = (h * jax.nn.sigmoid(h) * u).astype(jnp.bfloat16)
    shared = lax.dot_general(act, ws2_ref[...].astype(jnp.bfloat16),
                             (((1,), (1,)), ((), ())),
                             preferred_element_type=jnp.float32)
    g = jax.nn.sigmoid(lax.dot_general(x, sg_ref[...],
                                       (((1,), (1,)), ((), ())),
                                       preferred_element_type=jnp.float32))
    out_ref[...] = g0_ref[...] + g1_ref[...] + g * shared


def kernel(hidden_states, gate_w, w1, w2, w3, ws1, ws2, ws3, shared_gate_w):
    B, S, D = hidden_states.shape
    x = hidden_states.reshape(-1, D)
    T = x.shape[0]
    E = NUM_EXPERTS

    d0, d1, w0, w1n, te, xb, ac, mt = pl.pallas_call(
        _router_dispatch,
        out_shape=[
            jax.ShapeDtypeStruct((T, 1), jnp.int32),
            jax.ShapeDtypeStruct((T, 1), jnp.int32),
            jax.ShapeDtypeStruct((T, 1), jnp.float32),
            jax.ShapeDtypeStruct((T, 1), jnp.float32),
            jax.ShapeDtypeStruct((NT, 1), jnp.int32),
            jax.ShapeDtypeStruct((NT, 1), jnp.int32),
            jax.ShapeDtypeStruct((NT, 1), jnp.int32),
            jax.ShapeDtypeStruct((16, 1), jnp.int32),
        ],
        scratch_shapes=[pltpu.VMEM((T, E), jnp.float32),
                        pltpu.VMEM((T, E), jnp.float32)],
    )(x, gate_w)
    d0 = d0.reshape(T)
    d1 = d1.reshape(T)
    w0 = w0.reshape(T)
    w1n = w1n.reshape(T)
    te = te.reshape(NT)
    xb = xb.reshape(NT)
    ac = ac.reshape(NT)
    mt = mt.reshape(16)

    mesh = plsc.VectorSubcoreMesh(core_axis_name="c", subcore_axis_name="s")
    xs, ws = pl.kernel(
        _sc_dispatch_gather,
        out_type=[
            jax.ShapeDtypeStruct((S_PAD, D), jnp.float32),
            jax.ShapeDtypeStruct((S_PAD,), jnp.float32),
        ],
        mesh=mesh,
        scratch_types=[
            pltpu.VMEM((T,), jnp.int32),
            pltpu.VMEM((T,), jnp.float32),
            pltpu.VMEM((S_PAD,), jnp.int32),
            pltpu.VMEM((S_PAD,), jnp.float32),
            pltpu.VMEM((16,), jnp.int32),
            pltpu.VMEM((S_PAD // NW // 4, D), jnp.float32),
            pltpu.VMEM((S_PAD // NW // 4, D), jnp.float32),
            pltpu.SemaphoreType.DMA,
            pltpu.SemaphoreType.DMA,
            pltpu.SemaphoreType.DMA,
            pltpu.SemaphoreType.DMA,
        ],
        compiler_params=pltpu.CompilerParams(needs_layout_passes=False),
    )(x, d0, d1, w0, w1n, mt)

    grid_spec = pltpu.PrefetchScalarGridSpec(
        num_scalar_prefetch=3,
        grid=(NT,),
        in_specs=[
            pl.BlockSpec((TM, D), lambda j, te, xb, ac: (xb[j], 0)),
            pl.BlockSpec((1, MOE_FF, D), lambda j, te, xb, ac: (te[j], 0, 0)),
            pl.BlockSpec((1, D, MOE_FF), lambda j, te, xb, ac: (te[j], 0, 0)),
            pl.BlockSpec((1, MOE_FF, D), lambda j, te, xb, ac: (te[j], 0, 0)),
            pl.BlockSpec((TM, 1), lambda j, te, xb, ac: (xb[j], 0)),
        ],
        out_specs=pl.BlockSpec((TM, D), lambda j, te, xb, ac: (j, 0)),
    )
    y = pl.pallas_call(
        _moe_mm,
        grid_spec=grid_spec,
        out_shape=jax.ShapeDtypeStruct((S_PAD, D), jnp.float32),
        compiler_params=pltpu.CompilerParams(
            dimension_semantics=("arbitrary",)),
    )(te, xb, ac, xs, w1, w2, w3, ws.reshape(S_PAD, 1))

    g0, g1 = pl.kernel(
        _sc_combine_gather,
        out_type=[
            jax.ShapeDtypeStruct((T, D), jnp.float32),
            jax.ShapeDtypeStruct((T, D), jnp.float32),
        ],
        mesh=mesh,
        scratch_types=[
            pltpu.VMEM((2 * (T // NW),), jnp.int32),
            pltpu.VMEM((32, D), jnp.float32),
            pltpu.VMEM((32, D), jnp.float32),
            pltpu.SemaphoreType.DMA,
            pltpu.SemaphoreType.DMA,
            pltpu.SemaphoreType.DMA,
            pltpu.SemaphoreType.DMA,
        ],
        compiler_params=pltpu.CompilerParams(needs_layout_passes=False),
    )(y, d0, d1)

    nt = T // 512
    out = pl.pallas_call(
        _shared_final,
        grid=(nt,),
        in_specs=[
            pl.BlockSpec((512, D), lambda t: (t, 0)),
            pl.BlockSpec((MOE_FF, D), lambda t: (0, 0)),
            pl.BlockSpec((D, MOE_FF), lambda t: (0, 0)),
            pl.BlockSpec((MOE_FF, D), lambda t: (0, 0)),
            pl.BlockSpec((1, D), lambda t: (0, 0)),
            pl.BlockSpec((512, D), lambda t: (t, 0)),
            pl.BlockSpec((512, D), lambda t: (t, 0)),
        ],
        out_specs=pl.BlockSpec((512, D), lambda t: (t, 0)),
        out_shape=jax.ShapeDtypeStruct((T, D), jnp.float32),
    )(x, ws1, ws2, ws3, shared_gate_w, g0, g1)

    return out.reshape(B, S, D)


# speedup vs baseline: 1.0083x; 1.0083x over previous
import functools

import jax
import jax.numpy as jnp
from jax import lax
from jax.experimental import pallas as pl
from jax.experimental.pallas import tpu as pltpu
from jax.experimental.pallas import tpu_sc as plsc

NUM_EXPERTS = 16
HIDDEN = 1024
MOE_FF = 512
TM = 128
NT = 48
S_PAD = NT * TM
NW = 32


def _router_dispatch(x_ref, gate_ref, d0_ref, d1_ref, w0_ref, w1_ref,
                     te_ref, xb_ref, ac_ref, mt_ref, r1_ref, r2_ref):
    T = x_ref.shape[0]
    E = NUM_EXPERTS
    x = x_ref[...]
    logits = lax.dot_general(x, gate_ref[...], (((1,), (1,)), ((), ())),
                             preferred_element_type=jnp.float32)
    m = jnp.max(logits, axis=1, keepdims=True)
    ex = jnp.exp(logits - m)
    sm = ex / jnp.sum(ex, axis=1, keepdims=True)
    lane = lax.broadcasted_iota(jnp.int32, sm.shape, 1)
    m1 = jnp.max(sm, axis=1, keepdims=True)
    i1 = jnp.min(jnp.where(sm == m1, lane, E), axis=1, keepdims=True)
    oh1 = (lane == i1).astype(jnp.float32)
    smm = jnp.where(lane == i1, -jnp.inf, sm)
    m2 = jnp.max(smm, axis=1, keepdims=True)
    i2 = jnp.min(jnp.where(smm == m2, lane, E), axis=1, keepdims=True)
    oh2 = (lane == i2).astype(jnp.float32)
    denom = m1 + m2
    w0_ref[...] = m1 / denom
    w1_ref[...] = m2 / denom

    CH = 128
    rr = lax.broadcasted_iota(jnp.int32, (CH, CH), 0)
    cc = lax.broadcasted_iota(jnp.int32, (CH, CH), 1)
    lex = (cc < rr).astype(jnp.float32)
    off1 = jnp.zeros((1, E), jnp.float32)
    off2 = jnp.zeros((1, E), jnp.float32)
    for c in range(T // CH):
        sl = slice(c * CH, (c + 1) * CH)
        o1c = oh1[sl, :]
        o2c = oh2[sl, :]
        r1_ref[sl, :] = off1 + lax.dot_general(
            lex, o1c, (((1,), (0,)), ((), ())),
            preferred_element_type=jnp.float32)
        r2_ref[sl, :] = off2 + lax.dot_general(
            lex, o2c, (((1,), (0,)), ((), ())),
            preferred_element_type=jnp.float32)
        off1 = off1 + jnp.sum(o1c, axis=0, keepdims=True)
        off2 = off2 + jnp.sum(o2c, axis=0, keepdims=True)
    counts1 = off1
    counts = off1 + off2
    tcnt = jnp.floor((counts + (TM - 1)) * (1.0 / TM))
    er = lax.broadcasted_iota(jnp.int32, (E, E), 0)
    ec = lax.broadcasted_iota(jnp.int32, (E, E), 1)
    strict_upper = (er < ec).astype(jnp.float32)
    ts = lax.dot_general(tcnt, strict_upper, (((1,), (0,)), ((), ())),
                         preferred_element_type=jnp.float32)
    pt_start = ts * TM
    total = jnp.sum(tcnt)

    d0 = jnp.sum(oh1 * (pt_start + r1_ref[...]), axis=1, keepdims=True)
    d1 = jnp.sum(oh2 * (pt_start + counts1 + r2_ref[...]), axis=1,
                 keepdims=True)
    d0_ref[...] = d0.astype(jnp.int32)
    d1_ref[...] = d1.astype(jnp.int32)

    jcol = lax.broadcasted_iota(jnp.int32, (NT, 1), 0).astype(jnp.float32)
    jj = jnp.minimum(jcol, total - 1.0)
    lane_e = lax.broadcasted_iota(jnp.int32, (NT, E), 1).astype(jnp.float32)
    in_e = ((jj >= ts) & (jj < ts + tcnt)).astype(jnp.float32)
    te_ref[...] = jnp.sum(lane_e * in_e, axis=1, keepdims=True).astype(jnp.int32)
    xb_ref[...] = jj.astype(jnp.int32)
    ac_ref[...] = (jcol < total).astype(jnp.int32)
    mt_ref[...] = jnp.full((16, 1), TM, jnp.float32).astype(jnp.int32) * (
        total.astype(jnp.int32))


def _sc_dispatch_gather(x_hbm, d0_hbm, d1_hbm, w0_hbm, w1_hbm, mt_hbm,
                        xs_hbm, ws_hbm,
                        dest_v, wval_v, tok_v, wsort_v, meta_v, buf0_v, buf1_v,
                        sem_i0, sem_i1, sem_o0, sem_o1):
    T = d0_hbm.shape[0]
    wid = lax.axis_index("s") * 2 + lax.axis_index("c")
    pltpu.sync_copy(mt_hbm, meta_v)

    zf = jnp.zeros((16,), jnp.float32)

    def zbody(i, _):
        tok_v[pl.ds(i * 16, 16)] = (i * 16 + lax.iota(jnp.int32, 16)) & (T - 1)
        wsort_v[pl.ds(i * 16, 16)] = zf
        return 0

    lax.fori_loop(0, S_PAD // 16, zbody, 0, unroll=4)

    for d_hbm, w_hbm in ((d0_hbm, w0_hbm), (d1_hbm, w1_hbm)):
        pltpu.sync_copy(d_hbm, dest_v)
        pltpu.sync_copy(w_hbm, wval_v)

        def sbody(j, _):
            idx = dest_v[pl.ds(j * 16, 16)]
            vals = j * 16 + lax.iota(jnp.int32, 16)
            plsc.store_scatter(tok_v, [idx], vals)
            wv = wval_v[pl.ds(j * 16, 16)]
            plsc.store_scatter(wsort_v, [idx], wv)
            return 0

        lax.fori_loop(0, T // 16, sbody, 0, unroll=4)

    rows = S_PAD // NW
    base = pl.multiple_of(wid * rows, 8)
    pltpu.sync_copy(wsort_v.at[pl.ds(base, rows)],
                    ws_hbm.at[pl.ds(base, rows)])
    total_rows = meta_v[pl.ds(0, 16)][0]

    @pl.when(base < total_rows)
    def _():
        CH = 48
        nchk = rows // CH
        bufs = (buf0_v, buf1_v)
        sin = (sem_i0, sem_i1)
        sout = (sem_o0, sem_o1)

        def sl(c):
            return pl.ds(base + c * CH, CH)

        incp = [None] * nchk
        outcp = [None] * nchk
        incp[0] = pltpu.async_copy(x_hbm.at[tok_v.at[sl(0)]], bufs[0], sin[0])
        for c in range(nchk):
            if c + 1 < nchk:
                if c >= 1:
                    outcp[c - 1].wait()
                incp[c + 1] = pltpu.async_copy(
                    x_hbm.at[tok_v.at[sl(c + 1)]], bufs[(c + 1) % 2],
                    sin[(c + 1) % 2])
            incp[c].wait()
            outcp[c] = pltpu.async_copy(bufs[c % 2], xs_hbm.at[sl(c)],
                                        sout[c % 2])
        outcp[nchk - 2].wait()
        outcp[nchk - 1].wait()


def _moe_mm(te_ref, xb_ref, ac_ref, xs_ref, w1_ref, w2_ref, w3_ref, ws_ref,
            y_ref):
    j = pl.program_id(0)

    @pl.when(ac_ref[j] == 1)
    def _():
        xb = xs_ref[...].astype(jnp.bfloat16)
        h = lax.dot_general(xb, w1_ref[0].astype(jnp.bfloat16),
                            (((1,), (1,)), ((), ())),
                            preferred_element_type=jnp.float32)
        u = lax.dot_general(xb, w3_ref[0].astype(jnp.bfloat16),
                            (((1,), (1,)), ((), ())),
                            preferred_element_type=jnp.float32)
        act = (h * jax.nn.sigmoid(h) * u).astype(jnp.bfloat16)
        y = lax.dot_general(act, w2_ref[0].astype(jnp.bfloat16),
                            (((1,), (1,)), ((), ())),
                            preferred_element_type=jnp.float32)
        y_ref[...] = y * ws_ref[...]


def _sc_combine_gather(y_hbm, d0_hbm, d1_hbm, g0_hbm, g1_hbm,
                       idx_v, buf0_v, buf1_v, sem_i0, sem_i1, sem_o0, sem_o1):
    T = d0_hbm.shape[0]
    wid = lax.axis_index("s") * 2 + lax.axis_index("c")
    tpw = T // NW
    base = pl.multiple_of(wid * tpw, 8)
    pltpu.sync_copy(d0_hbm.at[pl.ds(base, tpw)], idx_v.at[pl.ds(0, tpw)])
    pltpu.sync_copy(d1_hbm.at[pl.ds(base, tpw)], idx_v.at[pl.ds(tpw, tpw)])
    CH = 32
    nchk = 2 * tpw // CH
    bufs = (buf0_v, buf1_v)
    sin = (sem_i0, sem_i1)
    sout = (sem_o0, sem_o1)
    outs = []
    for k in range(nchk):
        g_hbm = g0_hbm if k < nchk // 2 else g1_hbm
        outs.append(g_hbm.at[pl.ds(base + (k % (nchk // 2)) * CH, CH)])

    incp = [None] * nchk
    outcp = [None] * nchk
    incp[0] = pltpu.async_copy(y_hbm.at[idx_v.at[pl.ds(0, CH)]],
                               bufs[0], sin[0])
    for c in range(nchk):
        if c + 1 < nchk:
            if c >= 1:
                outcp[c - 1].wait()
            incp[c + 1] = pltpu.async_copy(
                y_hbm.at[idx_v.at[pl.ds((c + 1) * CH, CH)]],
                bufs[(c + 1) % 2], sin[(c + 1) % 2])
        incp[c].wait()
        outcp[c] = pltpu.async_copy(bufs[c % 2], outs[c], sout[c % 2])
    outcp[nchk - 2].wait()
    outcp[nchk - 1].wait()


def _shared_mm(x_ref, ws1_ref, ws2_ref, ws3_ref, sg_ref, ss_ref):
    x = x_ref[...]
    xb = x.astype(jnp.bfloat16)
    h = lax.dot_general(xb, ws1_ref[...].astype(jnp.bfloat16),
                        (((1,), (1,)), ((), ())),
                        preferred_element_type=jnp.float32)
    u = lax.dot_general(xb, ws3_ref[...].astype(jnp.bfloat16),
                        (((1,), (1,)), ((), ())),
                        preferred_element_type=jnp.float32)
    act = (h * jax.nn.sigmoid(h) * u).astype(jnp.bfloat16)
    shared = lax.dot_general(act, ws2_ref[...].astype(jnp.bfloat16),
                             (((1,), (1,)), ((), ())),
                             preferred_element_type=jnp.float32)
    g = jax.nn.sigmoid(lax.dot_general(x, sg_ref[...],
                                       (((1,), (1,)), ((), ())),
                                       preferred_element_type=jnp.float32))
    ss_ref[...] = g * shared


def _final_add(ss_ref, g0_ref, g1_ref, out_ref):
    out_ref[...] = g0_ref[...] + g1_ref[...] + ss_ref[...]


def kernel(hidden_states, gate_w, w1, w2, w3, ws1, ws2, ws3, shared_gate_w):
    B, S, D = hidden_states.shape
    x = hidden_states.reshape(-1, D)
    T = x.shape[0]
    E = NUM_EXPERTS

    d0, d1, w0, w1n, te, xb, ac, mt = pl.pallas_call(
        _router_dispatch,
        out_shape=[
            jax.ShapeDtypeStruct((T, 1), jnp.int32),
            jax.ShapeDtypeStruct((T, 1), jnp.int32),
            jax.ShapeDtypeStruct((T, 1), jnp.float32),
            jax.ShapeDtypeStruct((T, 1), jnp.float32),
            jax.ShapeDtypeStruct((NT, 1), jnp.int32),
            jax.ShapeDtypeStruct((NT, 1), jnp.int32),
            jax.ShapeDtypeStruct((NT, 1), jnp.int32),
            jax.ShapeDtypeStruct((16, 1), jnp.int32),
        ],
        scratch_shapes=[pltpu.VMEM((T, E), jnp.float32),
                        pltpu.VMEM((T, E), jnp.float32)],
    )(x, gate_w)
    d0 = d0.reshape(T)
    d1 = d1.reshape(T)
    w0 = w0.reshape(T)
    w1n = w1n.reshape(T)
    te = te.reshape(NT)
    xb = xb.reshape(NT)
    ac = ac.reshape(NT)
    mt = mt.reshape(16)

    nt = T // 512
    ss = pl.pallas_call(
        _shared_mm,
        grid=(nt,),
        in_specs=[
            pl.BlockSpec((512, D), lambda t: (t, 0)),
            pl.BlockSpec((MOE_FF, D), lambda t: (0, 0)),
            pl.BlockSpec((D, MOE_FF), lambda t: (0, 0)),
            pl.BlockSpec((MOE_FF, D), lambda t: (0, 0)),
            pl.BlockSpec((1, D), lambda t: (0, 0)),
        ],
        out_specs=pl.BlockSpec((512, D), lambda t: (t, 0)),
        out_shape=jax.ShapeDtypeStruct((T, D), jnp.float32),
    )(x, ws1, ws2, ws3, shared_gate_w)

    mesh = plsc.VectorSubcoreMesh(core_axis_name="c", subcore_axis_name="s")
    xs, ws = pl.kernel(
        _sc_dispatch_gather,
        out_type=[
            jax.ShapeDtypeStruct((S_PAD, D), jnp.float32),
            jax.ShapeDtypeStruct((S_PAD,), jnp.float32),
        ],
        mesh=mesh,
        scratch_types=[
            pltpu.VMEM((T,), jnp.int32),
            pltpu.VMEM((T,), jnp.float32),
            pltpu.VMEM((S_PAD,), jnp.int32),
            pltpu.VMEM((S_PAD,), jnp.float32),
            pltpu.VMEM((16,), jnp.int32),
            pltpu.VMEM((S_PAD // NW // 4, D), jnp.float32),
            pltpu.VMEM((S_PAD // NW // 4, D), jnp.float32),
            pltpu.SemaphoreType.DMA,
            pltpu.SemaphoreType.DMA,
            pltpu.SemaphoreType.DMA,
            pltpu.SemaphoreType.DMA,
        ],
        compiler_params=pltpu.CompilerParams(needs_layout_passes=False),
    )(x, d0, d1, w0, w1n, mt)

    grid_spec = pltpu.PrefetchScalarGridSpec(
        num_scalar_prefetch=3,
        grid=(NT,),
        in_specs=[
            pl.BlockSpec((TM, D), lambda j, te, xb, ac: (xb[j], 0)),
            pl.BlockSpec((1, MOE_FF, D), lambda j, te, xb, ac: (te[j], 0, 0)),
            pl.BlockSpec((1, D, MOE_FF), lambda j, te, xb, ac: (te[j], 0, 0)),
            pl.BlockSpec((1, MOE_FF, D), lambda j, te, xb, ac: (te[j], 0, 0)),
            pl.BlockSpec((TM, 1), lambda j, te, xb, ac: (xb[j], 0)),
        ],
        out_specs=pl.BlockSpec((TM, D), lambda j, te, xb, ac: (j, 0)),
    )
    y = pl.pallas_call(
        _moe_mm,
        grid_spec=grid_spec,
        out_shape=jax.ShapeDtypeStruct((S_PAD, D), jnp.float32),
        compiler_params=pltpu.CompilerParams(
            dimension_semantics=("arbitrary",)),
    )(te, xb, ac, xs, w1, w2, w3, ws.reshape(S_PAD, 1))

    g0, g1 = pl.kernel(
        _sc_combine_gather,
        out_type=[
            jax.ShapeDtypeStruct((T, D), jnp.float32),
            jax.ShapeDtypeStruct((T, D), jnp.float32),
        ],
        mesh=mesh,
        scratch_types=[
            pltpu.VMEM((2 * (T // NW),), jnp.int32),
            pltpu.VMEM((32, D), jnp.float32),
            pltpu.VMEM((32, D), jnp.float32),
            pltpu.SemaphoreType.DMA,
            pltpu.SemaphoreType.DMA,
            pltpu.SemaphoreType.DMA,
            pltpu.SemaphoreType.DMA,
        ],
        compiler_params=pltpu.CompilerParams(needs_layout_passes=False),
    )(y, d0, d1)

    out = pl.pallas_call(
        _final_add,
        grid=(nt,),
        in_specs=[
            pl.BlockSpec((512, D), lambda t: (t, 0)),
            pl.BlockSpec((512, D), lambda t: (t, 0)),
            pl.BlockSpec((512, D), lambda t: (t, 0)),
        ],
        out_specs=pl.BlockSpec((512, D), lambda t: (t, 0)),
        out_shape=jax.ShapeDtypeStruct((T, D), jnp.float32),
    )(ss, g0, g1)

    return out.reshape(B, S, D)
